# baseline (speedup 1.0000x reference)
import functools

import jax
import jax.numpy as jnp
from jax.experimental import pallas as pl
from jax.experimental.pallas import tpu as pltpu

BATCH = 16384
NUM_FIELDS = 26
EMBED_DIM = 64
NUM_INTERACTIONS = 100
BT = 512


def _body(pairs_ref, b_ref, emb_ref, s_ref, w_ref, out_ref):
    wv = w_ref[0, :]

    acc = jnp.zeros((BT, EMBED_DIM), jnp.float32)
    for p in range(NUM_INTERACTIONS):
        i1 = pairs_ref[p, 0]
        i2 = pairs_ref[p, 1]
        e1 = emb_ref[:, i1, :]
        e2 = emb_ref[:, i2, :]
        sp = s_ref[:, p:p + 1]
        acc = acc + sp * (e1 * e2)
    out_ref[:, :] = jnp.sum(acc * wv[None, :], axis=1, keepdims=True) + b_ref[0]


@functools.partial(jax.jit, static_argnames=("interpret",))
def kernel(embeddings, selected_pairs, interaction_scores, W, b, interpret=False):
    grid = (BATCH // BT,)
    return pl.pallas_call(
        _body,
        grid=grid,
        in_specs=[
            pl.BlockSpec(memory_space=pltpu.SMEM),
            pl.BlockSpec(memory_space=pltpu.SMEM),
            pl.BlockSpec((BT, NUM_FIELDS, EMBED_DIM), lambda i: (i, 0, 0)),
            pl.BlockSpec((BT, NUM_INTERACTIONS), lambda i: (i, 0)),
            pl.BlockSpec((1, EMBED_DIM), lambda i: (0, 0)),
        ],
        out_specs=pl.BlockSpec((BT, 1), lambda i: (i, 0)),
        out_shape=jax.ShapeDtypeStruct((BATCH, 1), jnp.float32),
        interpret=interpret,
    )(selected_pairs, b, embeddings, interaction_scores, W)
